# baseline (speedup 1.0000x reference)
import functools

import jax
import jax.numpy as jnp
from jax.experimental import pallas as pl
from jax.experimental.pallas import tpu as pltpu

TEMP = 0.5


def _head_kernel(feat_ref, W1_ref, b1_ref, W2_ref, b2_ref, Wg1_ref, out_ref):
    f1 = jnp.maximum(
        jnp.dot(feat_ref[...], W1_ref[...], preferred_element_type=jnp.float32)
        + b1_ref[...], 0.0)
    f2 = jnp.dot(f1, W2_ref[...], preferred_element_type=jnp.float32) + b2_ref[...]
    out_ref[...] = jnp.dot(f2, Wg1_ref[...], preferred_element_type=jnp.float32)


def _adj_mm1_kernel(adj_ref, sup_ref, Wg2_ref, bg1_ref, out_ref):
    acc = jnp.dot(adj_ref[...], sup_ref[...],
                  preferred_element_type=jnp.float32)
    h = jnp.maximum(acc + bg1_ref[...], 0.0)
    out_ref[...] = jnp.dot(h, Wg2_ref[...], preferred_element_type=jnp.float32)


def _adj_mm2_loss_kernel(adj_ref, sup_ref, bg2_ref, Wp1_ref, bp1_ref, Wp2_ref,
                         bp2_ref, out_ref, zn_ref, acc_ref, darg_ref,
                         *, bm, nb, n, inv_temp):
    i = pl.program_id(0)

    @pl.when(i == 0)
    def _():
        acc_ref[...] = jnp.zeros_like(acc_ref)
        out_ref[...] = jnp.zeros((1, 1), jnp.float32)

    acc = jnp.dot(adj_ref[...], sup_ref[...],
                  preferred_element_type=jnp.float32)
    logits = acc + bg2_ref[...]
    t = jnp.dot(logits, Wp1_ref[...],
                preferred_element_type=jnp.float32) + bp1_ref[...]
    t = jnp.where(t > 0, t, jnp.exp(jnp.minimum(t, 0.0)) - 1.0)
    z1 = jnp.dot(t, Wp2_ref[...],
                 preferred_element_type=jnp.float32) + bp2_ref[...]
    nsq = jnp.sum(z1 * z1, axis=1, keepdims=True)
    zn = z1 / jnp.maximum(jnp.sqrt(nsq), 1e-12)
    zn_bf = zn.astype(jnp.bfloat16)
    zn_ref[pl.ds(i * bm, bm), :] = zn_bf
    darg_ref[pl.ds(i * bm, bm), :] = (
        jnp.sum(zn * zn, axis=1, keepdims=True) * inv_temp)

    def body(j, r1):
        znj = zn_ref[pl.ds(j * bm, bm), :]
        b = jax.lax.dot_general(
            zn_bf, znj, (((1,), (1,)), ((), ())),
            preferred_element_type=jnp.float32)
        e = jnp.exp(b * inv_temp)
        r1 = r1 + jnp.sum(e, axis=1, keepdims=True)
        cmask = jnp.where(j < i, 1.0, 0.0)
        ct = jnp.swapaxes(jnp.sum(e, axis=0, keepdims=True), 0, 1)
        acc_ref[pl.ds(j * bm, bm), :] += ct * cmask
        return r1

    r1 = jax.lax.fori_loop(0, i + 1, body, jnp.zeros((bm, 1), jnp.float32))
    acc_ref[pl.ds(i * bm, bm), :] += r1

    @pl.when(i == nb - 1)
    def _():
        darg = darg_ref[...]
        x1 = acc_ref[...] + jnp.exp(darg)
        total = jnp.sum(jnp.log(x1) - darg) * (1.0 / n)
        out_ref[...] = jnp.full((1, 1), total, jnp.float32)


def kernel(adj1, feat1, W1, b1, W2, b2, Wg1, bg1, Wg2, bg2, Wp1, bp1, Wp2,
           bp2):
    n = adj1.shape[0]
    in_dim = feat1.shape[1]
    hid = Wg1.shape[1]
    out_dim = Wp1.shape[1]

    b1r = b1.reshape(1, -1)
    b2r = b2.reshape(1, -1)
    bg1r = bg1.reshape(1, -1)
    bg2r = bg2.reshape(1, -1)
    bp1r = bp1.reshape(1, -1)
    bp2r = bp2.reshape(1, -1)

    bm_head = n // 5
    sup1 = pl.pallas_call(
        _head_kernel,
        grid=(5,),
        in_specs=[
            pl.BlockSpec((bm_head, in_dim), lambda i: (i, 0)),
            pl.BlockSpec((in_dim, 64), lambda i: (0, 0)),
            pl.BlockSpec((1, 64), lambda i: (0, 0)),
            pl.BlockSpec((64, 32), lambda i: (0, 0)),
            pl.BlockSpec((1, 32), lambda i: (0, 0)),
            pl.BlockSpec((32, hid), lambda i: (0, 0)),
        ],
        out_specs=pl.BlockSpec((bm_head, hid), lambda i: (i, 0)),
        out_shape=jax.ShapeDtypeStruct((n, hid), jnp.float32),
    )(feat1, W1, b1r, W2, b2r, Wg1)

    bm = n // 25
    mm_grid = (n // bm,)
    adj_specs = [
        pl.BlockSpec((bm, n), lambda i: (i, 0)),
        pl.BlockSpec((n, hid), lambda i: (0, 0)),
    ]
    mm_params = pltpu.CompilerParams(dimension_semantics=("arbitrary",))

    sup2 = pl.pallas_call(
        _adj_mm1_kernel,
        grid=mm_grid,
        in_specs=adj_specs + [
            pl.BlockSpec((hid, hid), lambda i: (0, 0)),
            pl.BlockSpec((1, hid), lambda i: (0, 0)),
        ],
        out_specs=pl.BlockSpec((bm, hid), lambda i: (i, 0)),
        out_shape=jax.ShapeDtypeStruct((n, hid), jnp.float32),
        compiler_params=mm_params,
    )(adj1, sup1, Wg2, bg1r)

    bm2 = n // 50
    nb2 = n // bm2
    total = pl.pallas_call(
        functools.partial(_adj_mm2_loss_kernel, bm=bm2, nb=nb2, n=n,
                          inv_temp=1.0 / TEMP),
        grid=(nb2,),
        in_specs=[
            pl.BlockSpec((bm2, n), lambda i: (i, 0)),
            pl.BlockSpec((n, hid), lambda i: (0, 0)),
            pl.BlockSpec((1, hid), lambda i: (0, 0)),
            pl.BlockSpec((hid, out_dim), lambda i: (0, 0)),
            pl.BlockSpec((1, out_dim), lambda i: (0, 0)),
            pl.BlockSpec((out_dim, hid), lambda i: (0, 0)),
            pl.BlockSpec((1, hid), lambda i: (0, 0)),
        ],
        out_specs=pl.BlockSpec((1, 1), lambda i: (0, 0)),
        out_shape=jax.ShapeDtypeStruct((1, 1), jnp.float32),
        scratch_shapes=[
            pltpu.VMEM((n, hid), jnp.bfloat16),
            pltpu.VMEM((n, 1), jnp.float32),
            pltpu.VMEM((n, 1), jnp.float32),
        ],
        compiler_params=mm_params,
    )(adj1, sup2, bg2r, Wp1, bp1r, Wp2, bp2r)

    return total[0, 0]


# speedup vs baseline: 1.8688x; 1.8688x over previous
import functools

import jax
import jax.numpy as jnp
from jax.experimental import pallas as pl
from jax.experimental.pallas import tpu as pltpu

TEMP = 0.5


def _head_kernel(feat_ref, W1_ref, b1_ref, W2_ref, b2_ref, Wg1_ref, out_ref):
    f1 = jnp.maximum(
        jnp.dot(feat_ref[...], W1_ref[...], preferred_element_type=jnp.float32)
        + b1_ref[...], 0.0)
    f2 = jnp.dot(f1, W2_ref[...], preferred_element_type=jnp.float32) + b2_ref[...]
    out_ref[...] = jnp.dot(f2, Wg1_ref[...], preferred_element_type=jnp.float32)


def _adj_mm1_kernel(adj_ref, sup_ref, Wg2_ref, bg1_ref, out_ref):
    acc = jnp.dot(adj_ref[...], sup_ref[...],
                  preferred_element_type=jnp.float32)
    h = jnp.maximum(acc + bg1_ref[...], 0.0)
    out_ref[...] = jnp.dot(h, Wg2_ref[...], preferred_element_type=jnp.float32)


def _adj_mm2_loss_kernel(adj_ref, sup_ref, bg2_ref, Wp1_ref, bp1_ref, Wp2_ref,
                         bp2_ref, out_ref, zn_ref, acc_ref, darg_ref,
                         *, bm, nb, n, inv_temp):
    i = pl.program_id(0)

    i = pl.program_id(0)

    @pl.when(i == 0)
    def _():
        acc_ref[...] = jnp.zeros_like(acc_ref)
        out_ref[...] = jnp.zeros((1, 1), jnp.float32)

    acc = jnp.dot(adj_ref[...], sup_ref[...],
                  preferred_element_type=jnp.float32)
    logits = acc + bg2_ref[...]
    t = jnp.dot(logits, Wp1_ref[...],
                preferred_element_type=jnp.float32) + bp1_ref[...]
    t = jnp.where(t > 0, t, jnp.exp(jnp.minimum(t, 0.0)) - 1.0)
    z1 = jnp.dot(t, Wp2_ref[...],
                 preferred_element_type=jnp.float32) + bp2_ref[...]
    nsq = jnp.sum(z1 * z1, axis=1, keepdims=True)
    zn = z1 / jnp.maximum(jnp.sqrt(nsq), 1e-12)
    zn_bf = zn.astype(jnp.bfloat16)
    zn_ref[pl.ds(i * bm, bm), :] = zn_bf
    dargs = jnp.sum(zn * zn, axis=1, keepdims=True) * inv_temp
    darg_ref[pl.ds(i, 1), :] = jnp.swapaxes(dargs, 0, 1)

    def body(j, r1):
        znj = zn_ref[pl.ds(j * bm, bm), :]
        b = jax.lax.dot_general(
            zn_bf, znj, (((1,), (1,)), ((), ())),
            preferred_element_type=jnp.float32)
        e = jnp.exp(b * inv_temp)
        r1 = r1 + jnp.sum(e, axis=1, keepdims=True)
        cmask = jnp.where(j < i, 1.0, 0.0)
        acc_ref[pl.ds(j, 1), :] += jnp.sum(e, axis=0, keepdims=True) * cmask
        return r1

    r1 = jax.lax.fori_loop(0, i + 1, body, jnp.zeros((bm, 1), jnp.float32))
    acc_ref[pl.ds(i, 1), :] += jnp.swapaxes(r1, 0, 1)

    @pl.when(i == nb - 1)
    def _():
        darg = darg_ref[...]
        x1 = acc_ref[...] + jnp.exp(darg)
        total = jnp.sum(jnp.log(x1) - darg) * (1.0 / n)
        out_ref[...] = jnp.full((1, 1), total, jnp.float32)


def kernel(adj1, feat1, W1, b1, W2, b2, Wg1, bg1, Wg2, bg2, Wp1, bp1, Wp2,
           bp2):
    n = adj1.shape[0]
    in_dim = feat1.shape[1]
    hid = Wg1.shape[1]
    out_dim = Wp1.shape[1]

    b1r = b1.reshape(1, -1)
    b2r = b2.reshape(1, -1)
    bg1r = bg1.reshape(1, -1)
    bg2r = bg2.reshape(1, -1)
    bp1r = bp1.reshape(1, -1)
    bp2r = bp2.reshape(1, -1)

    bm_head = n // 5
    sup1 = pl.pallas_call(
        _head_kernel,
        grid=(5,),
        in_specs=[
            pl.BlockSpec((bm_head, in_dim), lambda i: (i, 0)),
            pl.BlockSpec((in_dim, 64), lambda i: (0, 0)),
            pl.BlockSpec((1, 64), lambda i: (0, 0)),
            pl.BlockSpec((64, 32), lambda i: (0, 0)),
            pl.BlockSpec((1, 32), lambda i: (0, 0)),
            pl.BlockSpec((32, hid), lambda i: (0, 0)),
        ],
        out_specs=pl.BlockSpec((bm_head, hid), lambda i: (i, 0)),
        out_shape=jax.ShapeDtypeStruct((n, hid), jnp.float32),
    )(feat1, W1, b1r, W2, b2r, Wg1)

    bm = n // 25
    mm_grid = (n // bm,)
    adj_specs = [
        pl.BlockSpec((bm, n), lambda i: (i, 0)),
        pl.BlockSpec((n, hid), lambda i: (0, 0)),
    ]
    mm_params = pltpu.CompilerParams(dimension_semantics=("arbitrary",))

    sup2 = pl.pallas_call(
        _adj_mm1_kernel,
        grid=mm_grid,
        in_specs=adj_specs + [
            pl.BlockSpec((hid, hid), lambda i: (0, 0)),
            pl.BlockSpec((1, hid), lambda i: (0, 0)),
        ],
        out_specs=pl.BlockSpec((bm, hid), lambda i: (i, 0)),
        out_shape=jax.ShapeDtypeStruct((n, hid), jnp.float32),
        compiler_params=mm_params,
    )(adj1, sup1, Wg2, bg1r)

    bm2 = n // 25
    nb2 = n // bm2
    total = pl.pallas_call(
        functools.partial(_adj_mm2_loss_kernel, bm=bm2, nb=nb2, n=n,
                          inv_temp=1.0 / TEMP),
        grid=(nb2,),
        in_specs=[
            pl.BlockSpec((bm2, n), lambda i: (i, 0)),
            pl.BlockSpec((n, hid), lambda i: (0, 0)),
            pl.BlockSpec((1, hid), lambda i: (0, 0)),
            pl.BlockSpec((hid, out_dim), lambda i: (0, 0)),
            pl.BlockSpec((1, out_dim), lambda i: (0, 0)),
            pl.BlockSpec((out_dim, hid), lambda i: (0, 0)),
            pl.BlockSpec((1, hid), lambda i: (0, 0)),
        ],
        out_specs=pl.BlockSpec((1, 1), lambda i: (0, 0)),
        out_shape=jax.ShapeDtypeStruct((1, 1), jnp.float32),
        scratch_shapes=[
            pltpu.VMEM((n, hid), jnp.bfloat16),
            pltpu.VMEM((nb2, bm2), jnp.float32),
            pltpu.VMEM((nb2, bm2), jnp.float32),
        ],
        compiler_params=mm_params,
    )(adj1, sup2, bg2r, Wp1, bp1r, Wp2, bp2r)

    return total[0, 0]


# speedup vs baseline: 1.9936x; 1.0668x over previous
import functools

import jax
import jax.numpy as jnp
from jax.experimental import pallas as pl
from jax.experimental.pallas import tpu as pltpu

TEMP = 0.5


def _head_kernel(feat_ref, W1_ref, b1_ref, W2_ref, b2_ref, Wg1_ref, out_ref):
    f1 = jnp.maximum(
        jnp.dot(feat_ref[...], W1_ref[...], preferred_element_type=jnp.float32)
        + b1_ref[...], 0.0)
    f2 = jnp.dot(f1, W2_ref[...], preferred_element_type=jnp.float32) + b2_ref[...]
    out_ref[...] = jnp.dot(f2, Wg1_ref[...], preferred_element_type=jnp.float32)


def _adj_mm1_kernel(adj_ref, sup_ref, Wg2_ref, bg1_ref, out_ref):
    acc = jnp.dot(adj_ref[...], sup_ref[...],
                  preferred_element_type=jnp.float32)
    h = jnp.maximum(acc + bg1_ref[...], 0.0)
    out_ref[...] = jnp.dot(h, Wg2_ref[...], preferred_element_type=jnp.float32)


def _adj_mm2_kernel(adj_ref, sup_ref, bg2_ref, Wp1_ref, bp1_ref, Wp2_ref,
                    bp2_ref, out_ref):
    acc = jnp.dot(adj_ref[...], sup_ref[...],
                  preferred_element_type=jnp.float32)
    logits = acc + bg2_ref[...]
    t = jnp.dot(logits, Wp1_ref[...],
                preferred_element_type=jnp.float32) + bp1_ref[...]
    t = jnp.where(t > 0, t, jnp.exp(jnp.minimum(t, 0.0)) - 1.0)
    z1 = jnp.dot(t, Wp2_ref[...],
                 preferred_element_type=jnp.float32) + bp2_ref[...]
    norm = jnp.sqrt(jnp.sum(z1 * z1, axis=1, keepdims=True))
    out_ref[...] = (z1 / jnp.maximum(norm, 1e-12)).astype(jnp.bfloat16)


def _loss_kernel(znr_ref, znc_ref, out_ref, acc_ref, darg_ref,
                 *, bs, ns, n, inv_temp):
    i = pl.program_id(0)
    j = pl.program_id(1)

    @pl.when(jnp.logical_and(i == 0, j == 0))
    def _():
        acc_ref[...] = jnp.zeros_like(acc_ref)

    @pl.when(j <= i)
    def _():
        zr = znr_ref[...]
        zc = znc_ref[...]
        b = jax.lax.dot_general(
            zr, zc, (((1,), (1,)), ((), ())),
            preferred_element_type=jnp.float32)
        e = jnp.exp(b * inv_temp)
        rs = jnp.sum(e, axis=1, keepdims=True)
        acc_ref[pl.ds(i, 1), :] += jnp.swapaxes(rs, 0, 1)

        @pl.when(j < i)
        def _():
            ones_r = jnp.ones((1, bs), jnp.float32)
            cs = jax.lax.dot_general(
                ones_r, e, (((1,), (0,)), ((), ())),
                preferred_element_type=jnp.float32)
            acc_ref[pl.ds(j, 1), :] += cs

        @pl.when(j == i)
        def _():
            zr32 = zr.astype(jnp.float32)
            nsq = jnp.sum(zr32 * zr32, axis=1, keepdims=True)
            darg_ref[pl.ds(i, 1), :] = jnp.swapaxes(nsq, 0, 1) * inv_temp

    @pl.when(jnp.logical_and(i == ns - 1, j == ns - 1))
    def _():
        darg = darg_ref[...]
        x1 = acc_ref[...] + jnp.exp(darg)
        total = jnp.sum(jnp.log(x1) - darg) * (1.0 / n)
        out_ref[...] = jnp.full((1, 1), total, jnp.float32)


def kernel(adj1, feat1, W1, b1, W2, b2, Wg1, bg1, Wg2, bg2, Wp1, bp1, Wp2,
           bp2):
    n = adj1.shape[0]
    in_dim = feat1.shape[1]
    hid = Wg1.shape[1]
    out_dim = Wp1.shape[1]

    b1r = b1.reshape(1, -1)
    b2r = b2.reshape(1, -1)
    bg1r = bg1.reshape(1, -1)
    bg2r = bg2.reshape(1, -1)
    bp1r = bp1.reshape(1, -1)
    bp2r = bp2.reshape(1, -1)

    bm_head = n // 5
    sup1 = pl.pallas_call(
        _head_kernel,
        grid=(5,),
        in_specs=[
            pl.BlockSpec((bm_head, in_dim), lambda i: (i, 0)),
            pl.BlockSpec((in_dim, 64), lambda i: (0, 0)),
            pl.BlockSpec((1, 64), lambda i: (0, 0)),
            pl.BlockSpec((64, 32), lambda i: (0, 0)),
            pl.BlockSpec((1, 32), lambda i: (0, 0)),
            pl.BlockSpec((32, hid), lambda i: (0, 0)),
        ],
        out_specs=pl.BlockSpec((bm_head, hid), lambda i: (i, 0)),
        out_shape=jax.ShapeDtypeStruct((n, hid), jnp.float32),
    )(feat1, W1, b1r, W2, b2r, Wg1)

    bm = n // 25
    mm_grid = (n // bm,)
    adj_specs = [
        pl.BlockSpec((bm, n), lambda i: (i, 0)),
        pl.BlockSpec((n, hid), lambda i: (0, 0)),
    ]
    mm_params = pltpu.CompilerParams(dimension_semantics=("arbitrary",))

    sup2 = pl.pallas_call(
        _adj_mm1_kernel,
        grid=mm_grid,
        in_specs=adj_specs + [
            pl.BlockSpec((hid, hid), lambda i: (0, 0)),
            pl.BlockSpec((1, hid), lambda i: (0, 0)),
        ],
        out_specs=pl.BlockSpec((bm, hid), lambda i: (i, 0)),
        out_shape=jax.ShapeDtypeStruct((n, hid), jnp.float32),
        compiler_params=mm_params,
    )(adj1, sup1, Wg2, bg1r)

    zn = pl.pallas_call(
        _adj_mm2_kernel,
        grid=mm_grid,
        in_specs=adj_specs + [
            pl.BlockSpec((1, hid), lambda i: (0, 0)),
            pl.BlockSpec((hid, out_dim), lambda i: (0, 0)),
            pl.BlockSpec((1, out_dim), lambda i: (0, 0)),
            pl.BlockSpec((out_dim, hid), lambda i: (0, 0)),
            pl.BlockSpec((1, hid), lambda i: (0, 0)),
        ],
        out_specs=pl.BlockSpec((bm, hid), lambda i: (i, 0)),
        out_shape=jax.ShapeDtypeStruct((n, hid), jnp.bfloat16),
        compiler_params=mm_params,
    )(adj1, sup2, bg2r, Wp1, bp1r, Wp2, bp2r)

    bs = n // 5
    ns = n // bs
    total = pl.pallas_call(
        functools.partial(_loss_kernel, bs=bs, ns=ns, n=n,
                          inv_temp=1.0 / TEMP),
        grid=(ns, ns),
        in_specs=[
            pl.BlockSpec((bs, hid), lambda i, j: (i, 0)),
            pl.BlockSpec((bs, hid), lambda i, j: (j, 0)),
        ],
        out_specs=pl.BlockSpec((1, 1), lambda i, j: (0, 0)),
        out_shape=jax.ShapeDtypeStruct((1, 1), jnp.float32),
        scratch_shapes=[
            pltpu.VMEM((ns, bs), jnp.float32),
            pltpu.VMEM((ns, bs), jnp.float32),
        ],
        compiler_params=pltpu.CompilerParams(
            dimension_semantics=("arbitrary", "arbitrary")),
    )(zn, zn)

    return total[0, 0]


# speedup vs baseline: 2.3298x; 1.1686x over previous
import functools

import jax
import jax.numpy as jnp
from jax.experimental import pallas as pl
from jax.experimental.pallas import tpu as pltpu

TEMP = 0.5


def _head_kernel(feat_ref, W1_ref, b1_ref, W2_ref, b2_ref, Wg1_ref, out_ref):
    f1 = jnp.maximum(
        jnp.dot(feat_ref[...], W1_ref[...], preferred_element_type=jnp.float32)
        + b1_ref[...], 0.0)
    f2 = jnp.dot(f1, W2_ref[...], preferred_element_type=jnp.float32) + b2_ref[...]
    out_ref[...] = jnp.dot(f2, Wg1_ref[...], preferred_element_type=jnp.float32)


def _adj_mm1_kernel(adj_ref, sup_ref, Wg2_ref, bg1_ref, out_ref):
    acc = jnp.dot(adj_ref[...], sup_ref[...],
                  preferred_element_type=jnp.float32)
    h = jnp.maximum(acc + bg1_ref[...], 0.0)
    out_ref[...] = jnp.dot(h, Wg2_ref[...], preferred_element_type=jnp.float32)


def _mm2_loss_kernel(adj_ref, sup_ref, bg2_ref, Wp1_ref, bp1_ref, Wp2_ref,
                     bp2_ref, out_ref, zn_ref, accs_ref, accf_ref, darg_ref,
                     *, bm, nb, ns, bs, n, scale):
    i = pl.program_id(0)
    spr = nb // ns

    @pl.when(i == 0)
    def _():
        accs_ref[...] = jnp.zeros_like(accs_ref)
        accf_ref[...] = jnp.zeros_like(accf_ref)
        darg_ref[...] = jnp.zeros_like(darg_ref)

    acc = jnp.dot(adj_ref[...], sup_ref[...],
                  preferred_element_type=jnp.float32)
    logits = acc + bg2_ref[...]
    t = jnp.dot(logits, Wp1_ref[...],
                preferred_element_type=jnp.float32) + bp1_ref[...]
    t = jnp.where(t > 0, t, jnp.exp(jnp.minimum(t, 0.0)) - 1.0)
    z1 = jnp.dot(t, Wp2_ref[...],
                 preferred_element_type=jnp.float32) + bp2_ref[...]
    norm = jnp.sqrt(jnp.sum(z1 * z1, axis=1, keepdims=True))
    zn = (z1 * (scale / jnp.maximum(norm, 1e-12))).astype(jnp.bfloat16)
    zn_ref[pl.ds(i * bm, bm), :] = zn
    zn32 = zn.astype(jnp.float32)
    dgrow = jnp.swapaxes(
        jnp.sum(zn32 * zn32, axis=1, keepdims=True), 0, 1)

    def wide_body(c, r1):
        zc = zn_ref[pl.ds(c * bs, bs), :]
        b = jax.lax.dot_general(
            zn, zc, (((1,), (1,)), ((), ())),
            preferred_element_type=jnp.float32)
        e = jnp.exp(b)
        r1 = r1 + jnp.sum(e, axis=1, keepdims=True)
        cs = jnp.sum(e, axis=0, keepdims=True)
        accs_ref[pl.ds(c, 1), :] += cs
        return r1

    r1 = jax.lax.fori_loop(0, i // spr, wide_body,
                           jnp.zeros((bm, 1), jnp.float32))

    band0 = (i // spr) * bs
    for k in range(spr):
        @pl.when(i % spr == k)
        def _(k=k):
            w = (k + 1) * bm
            zc = zn_ref[pl.ds(band0, w), :]
            b = jax.lax.dot_general(
                zn, zc, (((1,), (1,)), ((), ())),
                preferred_element_type=jnp.float32)
            e = jnp.exp(b)
            r1k = r1 + jnp.sum(e, axis=1, keepdims=True)
            rsp = jnp.pad(jnp.swapaxes(r1k, 0, 1),
                          ((0, 0), (k * bm, (spr - 1 - k) * bm)))
            accf_ref[pl.ds(i // spr, 1), :] += rsp
            dgp = jnp.pad(dgrow, ((0, 0), (k * bm, (spr - 1 - k) * bm)))
            darg_ref[pl.ds(i // spr, 1), :] += dgp
            if k > 0:
                cs = jnp.sum(e[:, :k * bm], axis=0, keepdims=True)
                csp = jnp.pad(cs, ((0, 0), (0, bs - k * bm)))
                accs_ref[pl.ds(i // spr, 1), :] += csp

    @pl.when(i == nb - 1)
    def _():
        darg = darg_ref[...]
        x1 = accf_ref[...] + accs_ref[...] + jnp.exp(darg)
        total = jnp.sum(jnp.log(x1) - darg) * (1.0 / n)
        out_ref[...] = jnp.full((1, 1), total, jnp.float32)


def kernel(adj1, feat1, W1, b1, W2, b2, Wg1, bg1, Wg2, bg2, Wp1, bp1, Wp2,
           bp2):
    n = adj1.shape[0]
    in_dim = feat1.shape[1]
    hid = Wg1.shape[1]
    out_dim = Wp1.shape[1]

    b1r = b1.reshape(1, -1)
    b2r = b2.reshape(1, -1)
    bg1r = bg1.reshape(1, -1)
    bg2r = bg2.reshape(1, -1)
    bp1r = bp1.reshape(1, -1)
    bp2r = bp2.reshape(1, -1)

    bm_head = n // 5
    sup1 = pl.pallas_call(
        _head_kernel,
        grid=(5,),
        in_specs=[
            pl.BlockSpec((bm_head, in_dim), lambda i: (i, 0)),
            pl.BlockSpec((in_dim, 64), lambda i: (0, 0)),
            pl.BlockSpec((1, 64), lambda i: (0, 0)),
            pl.BlockSpec((64, 32), lambda i: (0, 0)),
            pl.BlockSpec((1, 32), lambda i: (0, 0)),
            pl.BlockSpec((32, hid), lambda i: (0, 0)),
        ],
        out_specs=pl.BlockSpec((bm_head, hid), lambda i: (i, 0)),
        out_shape=jax.ShapeDtypeStruct((n, hid), jnp.float32),
    )(feat1, W1, b1r, W2, b2r, Wg1)

    bm = n // 25
    nb = 25
    ns = 5
    bs = (nb // ns) * bm
    mm_grid = (nb,)
    adj_specs = [
        pl.BlockSpec((bm, n), lambda i: (i, 0)),
        pl.BlockSpec((n, hid), lambda i: (0, 0)),
    ]
    mm_params = pltpu.CompilerParams(dimension_semantics=("arbitrary",))

    sup2 = pl.pallas_call(
        _adj_mm1_kernel,
        grid=mm_grid,
        in_specs=adj_specs + [
            pl.BlockSpec((hid, hid), lambda i: (0, 0)),
            pl.BlockSpec((1, hid), lambda i: (0, 0)),
        ],
        out_specs=pl.BlockSpec((bm, hid), lambda i: (i, 0)),
        out_shape=jax.ShapeDtypeStruct((n, hid), jnp.float32),
        compiler_params=mm_params,
    )(adj1, sup1, Wg2, bg1r)

    total = pl.pallas_call(
        functools.partial(_mm2_loss_kernel, bm=bm, nb=nb, ns=ns, bs=bs, n=n,
                          scale=(1.0 / TEMP) ** 0.5),
        grid=mm_grid,
        in_specs=adj_specs + [
            pl.BlockSpec((1, hid), lambda i: (0, 0)),
            pl.BlockSpec((hid, out_dim), lambda i: (0, 0)),
            pl.BlockSpec((1, out_dim), lambda i: (0, 0)),
            pl.BlockSpec((out_dim, hid), lambda i: (0, 0)),
            pl.BlockSpec((1, hid), lambda i: (0, 0)),
        ],
        out_specs=pl.BlockSpec((1, 1), lambda i: (0, 0)),
        out_shape=jax.ShapeDtypeStruct((1, 1), jnp.float32),
        scratch_shapes=[
            pltpu.VMEM((n, hid), jnp.bfloat16),
            pltpu.VMEM((ns, bs), jnp.float32),
            pltpu.VMEM((ns, bs), jnp.float32),
            pltpu.VMEM((ns, bs), jnp.float32),
        ],
        compiler_params=mm_params,
    )(adj1, sup2, bg2r, Wp1, bp1r, Wp2, bp2r)

    return total[0, 0]


# speedup vs baseline: 2.3994x; 1.0299x over previous
import functools

import jax
import jax.numpy as jnp
from jax.experimental import pallas as pl
from jax.experimental.pallas import tpu as pltpu

TEMP = 0.5


def _adj_mm1_kernel(adj_ref, feat_ref, W1_ref, b1_ref, W2_ref, b2_ref,
                    Wg1_ref, Wg2_ref, bg1_ref, out_ref, sup_ref):
    i = pl.program_id(0)

    @pl.when(i == 0)
    def _():
        f1 = jnp.maximum(
            jnp.dot(feat_ref[...], W1_ref[...],
                    preferred_element_type=jnp.float32) + b1_ref[...], 0.0)
        f2 = jnp.dot(f1, W2_ref[...],
                     preferred_element_type=jnp.float32) + b2_ref[...]
        sup_ref[...] = jnp.dot(f2, Wg1_ref[...],
                               preferred_element_type=jnp.float32)

    acc = jnp.dot(adj_ref[...], sup_ref[...],
                  preferred_element_type=jnp.float32)
    h = jnp.maximum(acc + bg1_ref[...], 0.0)
    out_ref[...] = jnp.dot(h, Wg2_ref[...], preferred_element_type=jnp.float32)


def _mm2_loss_kernel(adj_ref, sup_ref, bg2_ref, Wp1_ref, bp1_ref, Wp2_ref,
                     bp2_ref, out_ref, zn_ref, accs_ref, accf_ref, darg_ref,
                     *, bm, nb, ns, bs, n, scale):
    i = pl.program_id(0)
    spr = nb // ns

    @pl.when(i == 0)
    def _():
        accs_ref[...] = jnp.zeros_like(accs_ref)
        accf_ref[...] = jnp.zeros_like(accf_ref)
        darg_ref[...] = jnp.zeros_like(darg_ref)

    acc = jnp.dot(adj_ref[...], sup_ref[...],
                  preferred_element_type=jnp.float32)
    logits = acc + bg2_ref[...]
    t = jnp.dot(logits, Wp1_ref[...],
                preferred_element_type=jnp.float32) + bp1_ref[...]
    t = jnp.where(t > 0, t, jnp.exp(jnp.minimum(t, 0.0)) - 1.0)
    z1 = jnp.dot(t, Wp2_ref[...],
                 preferred_element_type=jnp.float32) + bp2_ref[...]
    norm = jnp.sqrt(jnp.sum(z1 * z1, axis=1, keepdims=True))
    zn = (z1 * (scale / jnp.maximum(norm, 1e-12))).astype(jnp.bfloat16)
    zn_ref[pl.ds(i * bm, bm), :] = zn
    zn32 = zn.astype(jnp.float32)
    dgrow = jnp.swapaxes(
        jnp.sum(zn32 * zn32, axis=1, keepdims=True), 0, 1)

    def wide_body(c, r1):
        zc = zn_ref[pl.ds(c * bs, bs), :]
        b = jax.lax.dot_general(
            zn, zc, (((1,), (1,)), ((), ())),
            preferred_element_type=jnp.float32)
        e = jnp.exp(b)
        r1 = r1 + jnp.sum(e, axis=1, keepdims=True)
        cs = jnp.sum(e, axis=0, keepdims=True)
        accs_ref[pl.ds(c, 1), :] += cs
        return r1

    r1 = jax.lax.fori_loop(0, i // spr, wide_body,
                           jnp.zeros((bm, 1), jnp.float32))

    band0 = (i // spr) * bs
    for k in range(spr):
        @pl.when(i % spr == k)
        def _(k=k):
            w = (k + 1) * bm
            zc = zn_ref[pl.ds(band0, w), :]
            b = jax.lax.dot_general(
                zn, zc, (((1,), (1,)), ((), ())),
                preferred_element_type=jnp.float32)
            e = jnp.exp(b)
            r1k = r1 + jnp.sum(e, axis=1, keepdims=True)
            rsp = jnp.pad(jnp.swapaxes(r1k, 0, 1),
                          ((0, 0), (k * bm, (spr - 1 - k) * bm)))
            accf_ref[pl.ds(i // spr, 1), :] += rsp
            dgp = jnp.pad(dgrow, ((0, 0), (k * bm, (spr - 1 - k) * bm)))
            darg_ref[pl.ds(i // spr, 1), :] += dgp
            if k > 0:
                cs = jnp.sum(e[:, :k * bm], axis=0, keepdims=True)
                csp = jnp.pad(cs, ((0, 0), (0, bs - k * bm)))
                accs_ref[pl.ds(i // spr, 1), :] += csp

    @pl.when(i == nb - 1)
    def _():
        darg = darg_ref[...]
        x1 = accf_ref[...] + accs_ref[...] + jnp.exp(darg)
        total = jnp.sum(jnp.log(x1) - darg) * (1.0 / n)
        out_ref[...] = jnp.full((1, 1), total, jnp.float32)


def kernel(adj1, feat1, W1, b1, W2, b2, Wg1, bg1, Wg2, bg2, Wp1, bp1, Wp2,
           bp2):
    n = adj1.shape[0]
    in_dim = feat1.shape[1]
    hid = Wg1.shape[1]
    out_dim = Wp1.shape[1]

    b1r = b1.reshape(1, -1)
    b2r = b2.reshape(1, -1)
    bg1r = bg1.reshape(1, -1)
    bg2r = bg2.reshape(1, -1)
    bp1r = bp1.reshape(1, -1)
    bp2r = bp2.reshape(1, -1)

    bm = n // 25
    nb = 25
    ns = 5
    bs = (nb // ns) * bm
    mm_grid = (nb,)
    adj_specs = [
        pl.BlockSpec((bm, n), lambda i: (i, 0)),
        pl.BlockSpec((n, hid), lambda i: (0, 0)),
    ]
    mm_params = pltpu.CompilerParams(dimension_semantics=("arbitrary",))

    sup2 = pl.pallas_call(
        _adj_mm1_kernel,
        grid=mm_grid,
        in_specs=[
            pl.BlockSpec((bm, n), lambda i: (i, 0)),
            pl.BlockSpec((n, in_dim), lambda i: (0, 0)),
            pl.BlockSpec((in_dim, 64), lambda i: (0, 0)),
            pl.BlockSpec((1, 64), lambda i: (0, 0)),
            pl.BlockSpec((64, 32), lambda i: (0, 0)),
            pl.BlockSpec((1, 32), lambda i: (0, 0)),
            pl.BlockSpec((32, hid), lambda i: (0, 0)),
            pl.BlockSpec((hid, hid), lambda i: (0, 0)),
            pl.BlockSpec((1, hid), lambda i: (0, 0)),
        ],
        out_specs=pl.BlockSpec((bm, hid), lambda i: (i, 0)),
        out_shape=jax.ShapeDtypeStruct((n, hid), jnp.float32),
        scratch_shapes=[pltpu.VMEM((n, hid), jnp.float32)],
        compiler_params=mm_params,
    )(adj1, feat1, W1, b1r, W2, b2r, Wg1, Wg2, bg1r)

    total = pl.pallas_call(
        functools.partial(_mm2_loss_kernel, bm=bm, nb=nb, ns=ns, bs=bs, n=n,
                          scale=(1.0 / TEMP) ** 0.5),
        grid=mm_grid,
        in_specs=adj_specs + [
            pl.BlockSpec((1, hid), lambda i: (0, 0)),
            pl.BlockSpec((hid, out_dim), lambda i: (0, 0)),
            pl.BlockSpec((1, out_dim), lambda i: (0, 0)),
            pl.BlockSpec((out_dim, hid), lambda i: (0, 0)),
            pl.BlockSpec((1, hid), lambda i: (0, 0)),
        ],
        out_specs=pl.BlockSpec((1, 1), lambda i: (0, 0)),
        out_shape=jax.ShapeDtypeStruct((1, 1), jnp.float32),
        scratch_shapes=[
            pltpu.VMEM((n, hid), jnp.bfloat16),
            pltpu.VMEM((ns, bs), jnp.float32),
            pltpu.VMEM((ns, bs), jnp.float32),
            pltpu.VMEM((ns, bs), jnp.float32),
        ],
        compiler_params=mm_params,
    )(adj1, sup2, bg2r, Wp1, bp1r, Wp2, bp2r)

    return total[0, 0]
